# baseline (speedup 1.0000x reference)
import functools

import jax
import jax.numpy as jnp
from jax import lax
from jax.experimental import pallas as pl
from jax.experimental.pallas import tpu as pltpu
from jax.experimental.pallas import tpu_sc as plsc

D_EMBED = 64
D_PAD = 128
NUM_WORKERS = 32
CHUNK = 128
NBUF = 4
LOOKAHEAD = 2


def _emb_body(b_per_w, n_chunk, idx_hbm, table_hbm, out_hbm, *scratch):
    idx_bufs = scratch[0:NBUF]
    row_bufs = scratch[NBUF:2 * NBUF]
    gsems = scratch[2 * NBUF:3 * NBUF]
    wsems = scratch[3 * NBUF:4 * NBUF]

    wid = lax.axis_index("s") * 2 + lax.axis_index("c")
    base = wid * b_per_w

    def _out_slice(g):
        return out_hbm.at[pl.ds(base + g * CHUNK, CHUNK)]

    def _stage(g, sb):
        pltpu.sync_copy(idx_hbm.at[pl.ds(base + g * CHUNK, CHUNK)], idx_bufs[sb])
        pltpu.async_copy(table_hbm.at[idx_bufs[sb]], row_bufs[sb], gsems[sb])

    for j in range(LOOKAHEAD):
        _stage(j, j)

    def _step(go, _):
        for b in range(NBUF):
            g = go + b
            sb = (b + LOOKAHEAD) % NBUF
            s = g + LOOKAHEAD

            @pl.when(s < n_chunk)
            def _():
                @pl.when(s >= NBUF)
                def _():
                    pltpu.make_async_copy(
                        row_bufs[sb], _out_slice(s - NBUF), wsems[sb]
                    ).wait()

                _stage(s, sb)

            pltpu.make_async_copy(
                table_hbm.at[idx_bufs[b]], row_bufs[b], gsems[b]
            ).wait()
            pltpu.async_copy(row_bufs[b], _out_slice(g), wsems[b])
        return _

    lax.fori_loop(0, n_chunk // NBUF, lambda i, c: _step(i * NBUF, c), None,
                  unroll=False)

    for j in range(NBUF):
        g = n_chunk - NBUF + j
        pltpu.make_async_copy(
            row_bufs[g % NBUF], _out_slice(g), wsems[g % NBUF]
        ).wait()


def kernel(inp, table):
    batch, hist = inp.shape
    n = batch * hist
    assert n % (NUM_WORKERS * CHUNK * NBUF) == 0
    b_per_w = n // NUM_WORKERS
    n_chunk = b_per_w // CHUNK

    table_padded = jnp.pad(table, ((0, 0), (0, D_PAD - D_EMBED)))
    flat_idx = inp.reshape(n)
    mesh = plsc.VectorSubcoreMesh(core_axis_name="c", subcore_axis_name="s")

    scratch = (
        [pltpu.VMEM((CHUNK,), jnp.int32) for _ in range(NBUF)]
        + [pltpu.VMEM((CHUNK, D_PAD), jnp.float32) for _ in range(NBUF)]
        + [pltpu.SemaphoreType.DMA for _ in range(2 * NBUF)]
    )
    grab = pl.kernel(
        functools.partial(_emb_body, b_per_w, n_chunk),
        mesh=mesh,
        compiler_params=pltpu.CompilerParams(use_tc_tiling_on_sc=True),
        out_type=jax.ShapeDtypeStruct((n, D_PAD), jnp.float32),
        scratch_types=scratch,
    )
    out = grab(flat_idx, table_padded)
    return out[:, :D_EMBED].reshape(batch, hist, D_EMBED)


# speedup vs baseline: 1.5317x; 1.5317x over previous
import functools

import jax
import jax.numpy as jnp
from jax import lax
from jax.experimental import pallas as pl
from jax.experimental.pallas import tpu as pltpu
from jax.experimental.pallas import tpu_sc as plsc

D_EMBED = 64
HIST = 50
HIST_PAD = 56
D_PAD = 128
NUM_WORKERS = 32
B_CHUNK = 8
CHUNK = B_CHUNK * HIST
NBUF = 4
LOOKAHEAD = 2


def _emb_body(b_per_w, n_chunk, idx_hbm, table_hbm, out_hbm, *scratch):
    idx_bufs = scratch[0:NBUF]
    row_bufs = scratch[NBUF:2 * NBUF]
    gsems = scratch[2 * NBUF:3 * NBUF]
    wsems = scratch[3 * NBUF:4 * NBUF]

    wid = lax.axis_index("s") * 2 + lax.axis_index("c")
    base = wid * b_per_w
    b_base = wid * (b_per_w // HIST)

    def _write(g, rows_v, wsem):
        for i in range(B_CHUNK):
            pltpu.async_copy(
                rows_v.at[pl.ds(i * HIST, HIST)],
                out_hbm.at[b_base + g * B_CHUNK + i,
                           pl.ds(0, HIST), pl.ds(0, D_EMBED)],
                wsem)

    def _wait_write(g, rows_v, wsem):
        for i in range(B_CHUNK):
            pltpu.make_async_copy(
                rows_v.at[pl.ds(i * HIST, HIST)],
                out_hbm.at[b_base + g * B_CHUNK + i,
                           pl.ds(0, HIST), pl.ds(0, D_EMBED)],
                wsem).wait()

    def _stage(g, sb):
        pltpu.sync_copy(idx_hbm.at[pl.ds(base + g * CHUNK, CHUNK)], idx_bufs[sb])
        pltpu.async_copy(table_hbm.at[idx_bufs[sb]], row_bufs[sb], gsems[sb])

    for j in range(LOOKAHEAD):
        _stage(j, j)

    def _step(go, _):
        for b in range(NBUF):
            g = go + b
            sb = (b + LOOKAHEAD) % NBUF
            s = g + LOOKAHEAD

            @pl.when(s < n_chunk)
            def _():
                @pl.when(s >= NBUF)
                def _():
                    _wait_write(s - NBUF, row_bufs[sb], wsems[sb])

                _stage(s, sb)

            pltpu.make_async_copy(
                table_hbm.at[idx_bufs[b]], row_bufs[b], gsems[b]
            ).wait()
            _write(g, row_bufs[b], wsems[b])
        return _

    lax.fori_loop(0, n_chunk // NBUF, lambda i, c: _step(i * NBUF, c), None,
                  unroll=False)

    for j in range(NBUF):
        g = n_chunk - NBUF + j
        _wait_write(g, row_bufs[g % NBUF], wsems[g % NBUF])


def kernel(inp, table):
    batch, hist = inp.shape
    n = batch * hist
    assert n % (NUM_WORKERS * CHUNK * NBUF) == 0
    b_per_w = n // NUM_WORKERS
    n_chunk = b_per_w // CHUNK

    flat_idx = inp.reshape(n)
    mesh = plsc.VectorSubcoreMesh(core_axis_name="c", subcore_axis_name="s")

    scratch = (
        [pltpu.VMEM((CHUNK,), jnp.int32) for _ in range(NBUF)]
        + [pltpu.VMEM((CHUNK, D_EMBED), jnp.float32) for _ in range(NBUF)]
        + [pltpu.SemaphoreType.DMA for _ in range(2 * NBUF)]
    )
    grab = pl.kernel(
        functools.partial(_emb_body, b_per_w, n_chunk),
        mesh=mesh,
        compiler_params=pltpu.CompilerParams(use_tc_tiling_on_sc=False),
        out_type=jax.ShapeDtypeStruct((batch, HIST_PAD, D_PAD), jnp.float32),
        scratch_types=scratch,
    )
    out = grab(flat_idx, table)
    return jax.lax.slice(out, (0, 0, 0), (batch, hist, D_EMBED))
